# baseline (speedup 1.0000x reference)
import functools

import jax
import jax.numpy as jnp
from jax import lax
from jax.experimental import pallas as pl
from jax.experimental.pallas import tpu as pltpu
from jax.experimental.pallas import tpu_sc as plsc

N = 10000
E_TOTAL = 320000
NFEAT = 128
NHID = 128
NCLASS = 47
DPAD = 128

NC, NS = 2, 16
NW = NC * NS
ECH = 50
CH_PER_W = E_TOTAL // (ECH * NW)
E_PER_W = ECH * CH_PER_W
NBUF = 5
NGRP = CH_PER_W // NBUF
N_PAD1 = 10240
ROWS1 = N_PAD1 // NS
ROWS_A = 624
ROWS_LAST = N - (NS - 1) * ROWS_A
OFF_LAST = (NS - 1) * ROWS_A


def _sc_mesh():
    return plsc.VectorSubcoreMesh(
        core_axis_name="c", subcore_axis_name="s", num_cores=NC, num_subcores=NS
    )


def _deg_kernel(dst2d, zeros1d):
    @functools.partial(
        pl.kernel,
        out_type=jax.ShapeDtypeStruct((NC, N_PAD1), jnp.float32),
        mesh=_sc_mesh(),
        scratch_types=[
            pltpu.VMEM((CH_PER_W, ECH), jnp.int32),
            pltpu.VMEM((64,), jnp.float32),
            pltpu.VMEM_SHARED((N_PAD1,), jnp.float32),
            [pltpu.SemaphoreType.DMA] * NBUF,
        ],
    )
    def k(dst_hbm, zeros_hbm, out_hbm, dst_v, ones_v, acc_sh, ssem):
        c = lax.axis_index("c")
        s = lax.axis_index("s")
        wid = s * NC + c
        r0 = s * ROWS1
        pltpu.sync_copy(
            zeros_hbm.at[pl.ds(r0, ROWS1)], acc_sh.at[pl.ds(r0, ROWS1)]
        )
        pltpu.sync_copy(dst_hbm.at[wid], dst_v)
        for j in range(4):
            ones_v[pl.ds(j * 16, 16)] = jnp.full((16,), 1.0, jnp.float32)
        plsc.subcore_barrier()

        def body(g, carry):
            descs = []
            for b in range(NBUF):
                i = g * NBUF + b
                descs.append(
                    pltpu.async_copy(
                        ones_v.at[pl.ds(0, ECH)], acc_sh.at[dst_v.at[i]],
                        ssem[b], add=True,
                    )
                )
            for d in descs:
                d.wait()
            return carry

        lax.fori_loop(0, NGRP, body, 0)
        plsc.subcore_barrier()
        pltpu.sync_copy(
            acc_sh.at[pl.ds(r0, ROWS1)],
            out_hbm.at[c].at[pl.ds(r0, ROWS1)],
        )

    return k(dst2d, zeros1d)


def _seg_sum(hs, src3d, dst3d, zeros2d, d):
    @functools.partial(
        pl.kernel,
        out_type=jax.ShapeDtypeStruct((NC, N, d), jnp.float32),
        mesh=_sc_mesh(),
        scratch_types=[
            [pltpu.VMEM((ECH,), jnp.int32)] * NBUF,
            [pltpu.VMEM((ECH,), jnp.int32)] * NBUF,
            [pltpu.VMEM((ECH,), jnp.int32)] * NBUF,
            [pltpu.VMEM((ECH, d), jnp.float32)] * NBUF,
            pltpu.VMEM_SHARED((N, d), jnp.float32),
            [pltpu.SemaphoreType.DMA] * NBUF,
            [pltpu.SemaphoreType.DMA] * NBUF,
            [pltpu.SemaphoreType.DMA] * NBUF,
            [pltpu.SemaphoreType.DMA] * NBUF,
        ],
    )
    def k(hs_hbm, src_hbm, dst_hbm, zeros_hbm, out_hbm, src_v, dst_va, dst_vb, rows_v, acc_sh, isem, jsem, gsem, ssem):
        c = lax.axis_index("c")
        s = lax.axis_index("s")
        wid = s * NC + c
        r0 = s * ROWS_A

        def _rowcopy(src_ref, dst_ref):
            @pl.when(s != NS - 1)
            def _():
                pltpu.sync_copy(
                    src_ref.at[pl.ds(r0, ROWS_A)], dst_ref.at[pl.ds(r0, ROWS_A)]
                )

            @pl.when(s == NS - 1)
            def _():
                pltpu.sync_copy(
                    src_ref.at[pl.ds(OFF_LAST, ROWS_LAST)],
                    dst_ref.at[pl.ds(OFF_LAST, ROWS_LAST)],
                )

        @pl.when(c == 0)
        def _():
            _rowcopy(hs_hbm, acc_sh)

        @pl.when(c != 0)
        def _():
            _rowcopy(zeros_hbm, acc_sh)

        plsc.subcore_barrier()

        def _issue_chunk(i, b, dst_slot):
            pltpu.async_copy(src_hbm.at[wid].at[i], src_v[b], isem[b])
            pltpu.async_copy(dst_hbm.at[wid].at[i], dst_slot[b], jsem[b])

        def _drain_scatter(b, dst_slot):
            pltpu.make_async_copy(
                rows_v[b], acc_sh.at[dst_slot[b]], ssem[b]
            ).wait()

        def _group(g, dst_cur, dst_nxt):
            gd = []
            for b in range(NBUF):
                @pl.when(g > 0)
                def _():
                    _drain_scatter(b, dst_nxt)

                pltpu.make_async_copy(
                    src_hbm.at[wid].at[g * NBUF + b], src_v[b], isem[b]
                ).wait()
                gd.append(
                    pltpu.async_copy(hs_hbm.at[src_v[b]], rows_v[b], gsem[b])
                )
            for b in range(NBUF):
                gd[b].wait()
                pltpu.make_async_copy(
                    dst_hbm.at[wid].at[g * NBUF + b], dst_cur[b], jsem[b]
                ).wait()
                pltpu.async_copy(
                    rows_v[b], acc_sh.at[dst_cur[b]], ssem[b], add=True
                )

                @pl.when(g < NGRP - 1)
                def _():
                    _issue_chunk((g + 1) * NBUF + b, b, dst_nxt)

        for b in range(NBUF):
            _issue_chunk(b, b, dst_va)

        def body(g, carry):
            @pl.when(g % 2 == 0)
            def _():
                _group(g, dst_va, dst_vb)

            @pl.when(g % 2 == 1)
            def _():
                _group(g, dst_vb, dst_va)

            return carry

        lax.fori_loop(0, NGRP, body, 0)
        for b in range(NBUF):
            _drain_scatter(b, dst_vb if (NGRP - 1) % 2 == 1 else dst_va)
        plsc.subcore_barrier()
        _rowcopy(acc_sh, out_hbm.at[c])

    return k(hs, src3d, dst3d, zeros2d)


_R = 10000


def _t1(deg, x, w1):
    def body(deg_ref, x_ref, w_ref, hs_ref, dinv_ref):
        d = deg_ref[0] + deg_ref[1] + 1.0
        dinv = lax.rsqrt(d)
        h = jnp.dot(x_ref[...], w_ref[...], preferred_element_type=jnp.float32)
        hs_ref[...] = h * dinv
        dinv_ref[...] = dinv

    return pl.pallas_call(
        body,
        grid=(N // _R,),
        in_specs=[
            pl.BlockSpec((NC, _R, 1), lambda i: (0, i, 0)),
            pl.BlockSpec((_R, NFEAT), lambda i: (i, 0)),
            pl.BlockSpec((NFEAT, NHID), lambda i: (0, 0)),
        ],
        out_specs=[
            pl.BlockSpec((_R, NHID), lambda i: (i, 0)),
            pl.BlockSpec((_R, 1), lambda i: (i, 0)),
        ],
        out_shape=[
            jax.ShapeDtypeStruct((N, NHID), jnp.float32),
            jax.ShapeDtypeStruct((N, 1), jnp.float32),
        ],
    )(deg, x, w1)


def _t_mid(dinv, agg, b, w, d_out):
    def body(dinv_ref, agg_ref, b_ref, w_ref, hs_ref):
        dinv = dinv_ref[...]
        a = (agg_ref[0] + agg_ref[1]) * dinv + b_ref[...]
        h_in = jnp.maximum(a, 0.0)
        h = jnp.dot(h_in, w_ref[...], preferred_element_type=jnp.float32)
        hs_ref[...] = h * dinv

    return pl.pallas_call(
        body,
        grid=(N // _R,),
        in_specs=[
            pl.BlockSpec((_R, 1), lambda i: (i, 0)),
            pl.BlockSpec((NC, _R, NHID), lambda i: (0, i, 0)),
            pl.BlockSpec((1, NHID), lambda i: (0, 0)),
            pl.BlockSpec((NHID, d_out), lambda i: (0, 0)),
        ],
        out_specs=pl.BlockSpec((_R, d_out), lambda i: (i, 0)),
        out_shape=jax.ShapeDtypeStruct((N, d_out), jnp.float32),
    )(dinv, agg, b, w)


def _t_final(dinv, agg, b):
    def body(dinv_ref, agg_ref, b_ref, out_ref):
        dinv = dinv_ref[...]
        z = (agg_ref[0] + agg_ref[1]) * dinv + b_ref[...]
        col = lax.broadcasted_iota(jnp.int32, z.shape, 1)
        z = jnp.where(col < NCLASS, z, -1e30)
        m = jnp.max(z, axis=1, keepdims=True)
        ez = jnp.where(col < NCLASS, jnp.exp(z - m), 0.0)
        lse = jnp.log(jnp.sum(ez, axis=1, keepdims=True))
        out_ref[...] = (z - m - lse)[:, :NCLASS]

    return pl.pallas_call(
        body,
        grid=(N // _R,),
        in_specs=[
            pl.BlockSpec((_R, 1), lambda i: (i, 0)),
            pl.BlockSpec((NC, _R, DPAD), lambda i: (0, i, 0)),
            pl.BlockSpec((1, DPAD), lambda i: (0, 0)),
        ],
        out_specs=pl.BlockSpec((_R, NCLASS), lambda i: (i, 0)),
        out_shape=jax.ShapeDtypeStruct((N, NCLASS), jnp.float32),
    )(dinv, agg, b)


def kernel(x, adj_t, W1, b1, W2, b2, W3, b3):
    src2d = adj_t[0].reshape(NW, CH_PER_W, ECH)
    dst2d = adj_t[1].reshape(NW, CH_PER_W, ECH)
    zeros1d = jnp.zeros((N_PAD1,), jnp.float32)
    zeros2d = jnp.zeros((N, NHID), jnp.float32)
    w3p = jnp.pad(W3, ((0, 0), (0, DPAD - NCLASS)))
    b3p = jnp.pad(b3, (0, DPAD - NCLASS)).reshape(1, DPAD)

    deg = _deg_kernel(dst2d, zeros1d).reshape(NC, N_PAD1, 1)
    hs1, dinv = _t1(deg, x, W1)
    agg1 = _seg_sum(hs1, src2d, dst2d, zeros2d, NHID)
    hs2 = _t_mid(dinv, agg1, b1.reshape(1, NHID), W2, NHID)
    agg2 = _seg_sum(hs2, src2d, dst2d, zeros2d, NHID)
    hs3 = _t_mid(dinv, agg2, b2.reshape(1, NHID), w3p, DPAD)
    agg3 = _seg_sum(hs3, src2d, dst2d, zeros2d, DPAD)
    return _t_final(dinv, agg3, b3p)


# speedup vs baseline: 1.0835x; 1.0835x over previous
import functools

import jax
import jax.numpy as jnp
from jax import lax
from jax.experimental import pallas as pl
from jax.experimental.pallas import tpu as pltpu
from jax.experimental.pallas import tpu_sc as plsc

N = 10000
E_TOTAL = 320000
NFEAT = 128
NHID = 128
NCLASS = 47
DPAD = 64

NC, NS = 2, 16
NW = NC * NS
ECH = 50
CH_PER_W = E_TOTAL // (ECH * NW)
E_PER_W = ECH * CH_PER_W
NBUF = 5
NGRP = CH_PER_W // NBUF
N_PAD1 = 10240
ROWS1 = N_PAD1 // NS
ROWS_A = 624
ROWS_LAST = N - (NS - 1) * ROWS_A
OFF_LAST = (NS - 1) * ROWS_A


def _sc_mesh():
    return plsc.VectorSubcoreMesh(
        core_axis_name="c", subcore_axis_name="s", num_cores=NC, num_subcores=NS
    )


def _deg_kernel(dst2d, zeros1d):
    @functools.partial(
        pl.kernel,
        out_type=jax.ShapeDtypeStruct((NC, N_PAD1), jnp.float32),
        mesh=_sc_mesh(),
        scratch_types=[
            pltpu.VMEM((CH_PER_W, ECH), jnp.int32),
            pltpu.VMEM((64,), jnp.float32),
            pltpu.VMEM_SHARED((N_PAD1,), jnp.float32),
            [pltpu.SemaphoreType.DMA] * NBUF,
        ],
    )
    def k(dst_hbm, zeros_hbm, out_hbm, dst_v, ones_v, acc_sh, ssem):
        c = lax.axis_index("c")
        s = lax.axis_index("s")
        wid = s * NC + c
        r0 = s * ROWS1
        pltpu.sync_copy(
            zeros_hbm.at[pl.ds(r0, ROWS1)], acc_sh.at[pl.ds(r0, ROWS1)]
        )
        pltpu.sync_copy(dst_hbm.at[wid], dst_v)
        for j in range(4):
            ones_v[pl.ds(j * 16, 16)] = jnp.full((16,), 1.0, jnp.float32)
        plsc.subcore_barrier()

        def body(g, carry):
            descs = []
            for b in range(NBUF):
                i = g * NBUF + b
                descs.append(
                    pltpu.async_copy(
                        ones_v.at[pl.ds(0, ECH)], acc_sh.at[dst_v.at[i]],
                        ssem[b], add=True,
                    )
                )
            for d in descs:
                d.wait()
            return carry

        lax.fori_loop(0, NGRP, body, 0)
        plsc.subcore_barrier()
        pltpu.sync_copy(
            acc_sh.at[pl.ds(r0, ROWS1)],
            out_hbm.at[c].at[pl.ds(r0, ROWS1)],
        )

    return k(dst2d, zeros1d)


def _seg_sum(hs, src3d, dst3d, zeros2d, d, tc_tiling=True):
    @functools.partial(
        pl.kernel,
        out_type=jax.ShapeDtypeStruct((NC, N, d), jnp.float32),
        mesh=_sc_mesh(),
        compiler_params=pltpu.CompilerParams(use_tc_tiling_on_sc=tc_tiling),
        scratch_types=[
            [pltpu.VMEM((ECH,), jnp.int32)] * NBUF,
            [pltpu.VMEM((ECH,), jnp.int32)] * NBUF,
            [pltpu.VMEM((ECH,), jnp.int32)] * NBUF,
            [pltpu.VMEM((ECH, d), jnp.float32)] * NBUF,
            pltpu.VMEM_SHARED((N, d), jnp.float32),
            [pltpu.SemaphoreType.DMA] * NBUF,
            [pltpu.SemaphoreType.DMA] * NBUF,
            [pltpu.SemaphoreType.DMA] * NBUF,
            [pltpu.SemaphoreType.DMA] * NBUF,
        ],
    )
    def k(hs_hbm, src_hbm, dst_hbm, zeros_hbm, out_hbm, src_v, dst_va, dst_vb, rows_v, acc_sh, isem, jsem, gsem, ssem):
        c = lax.axis_index("c")
        s = lax.axis_index("s")
        wid = s * NC + c
        r0 = s * ROWS_A

        def _rowcopy(src_ref, dst_ref):
            @pl.when(s != NS - 1)
            def _():
                pltpu.sync_copy(
                    src_ref.at[pl.ds(r0, ROWS_A)], dst_ref.at[pl.ds(r0, ROWS_A)]
                )

            @pl.when(s == NS - 1)
            def _():
                pltpu.sync_copy(
                    src_ref.at[pl.ds(OFF_LAST, ROWS_LAST)],
                    dst_ref.at[pl.ds(OFF_LAST, ROWS_LAST)],
                )

        @pl.when(c == 0)
        def _():
            _rowcopy(hs_hbm, acc_sh)

        @pl.when(c != 0)
        def _():
            _rowcopy(zeros_hbm, acc_sh)

        plsc.subcore_barrier()

        def _issue_chunk(i, b, dst_slot):
            pltpu.async_copy(src_hbm.at[wid].at[i], src_v[b], isem[b])
            pltpu.async_copy(dst_hbm.at[wid].at[i], dst_slot[b], jsem[b])

        def _drain_scatter(b, dst_slot):
            pltpu.make_async_copy(
                rows_v[b], acc_sh.at[dst_slot[b]], ssem[b]
            ).wait()

        def _group(g, dst_cur, dst_nxt):
            gd = []
            for b in range(NBUF):
                @pl.when(g > 0)
                def _():
                    _drain_scatter(b, dst_nxt)

                pltpu.make_async_copy(
                    src_hbm.at[wid].at[g * NBUF + b], src_v[b], isem[b]
                ).wait()
                gd.append(
                    pltpu.async_copy(hs_hbm.at[src_v[b]], rows_v[b], gsem[b])
                )
            for b in range(NBUF):
                gd[b].wait()
                pltpu.make_async_copy(
                    dst_hbm.at[wid].at[g * NBUF + b], dst_cur[b], jsem[b]
                ).wait()
                pltpu.async_copy(
                    rows_v[b], acc_sh.at[dst_cur[b]], ssem[b], add=True
                )

                @pl.when(g < NGRP - 1)
                def _():
                    _issue_chunk((g + 1) * NBUF + b, b, dst_nxt)

        for b in range(NBUF):
            _issue_chunk(b, b, dst_va)

        def body(g, carry):
            @pl.when(g % 2 == 0)
            def _():
                _group(g, dst_va, dst_vb)

            @pl.when(g % 2 == 1)
            def _():
                _group(g, dst_vb, dst_va)

            return carry

        lax.fori_loop(0, NGRP, body, 0)
        for b in range(NBUF):
            _drain_scatter(b, dst_vb if (NGRP - 1) % 2 == 1 else dst_va)
        plsc.subcore_barrier()
        _rowcopy(acc_sh, out_hbm.at[c])

    return k(hs, src3d, dst3d, zeros2d)


_R = 5000


def _t1(deg, x, w1):
    def body(deg_ref, x_ref, w_ref, hs_ref, dinv_ref):
        d = deg_ref[0] + deg_ref[1] + 1.0
        dinv = lax.rsqrt(d)
        h = jnp.dot(x_ref[...], w_ref[...], preferred_element_type=jnp.float32)
        hs_ref[...] = h * dinv
        dinv_ref[...] = dinv

    return pl.pallas_call(
        body,
        grid=(N // _R,),
        in_specs=[
            pl.BlockSpec((NC, _R, 1), lambda i: (0, i, 0)),
            pl.BlockSpec((_R, NFEAT), lambda i: (i, 0)),
            pl.BlockSpec((NFEAT, NHID), lambda i: (0, 0)),
        ],
        out_specs=[
            pl.BlockSpec((_R, NHID), lambda i: (i, 0)),
            pl.BlockSpec((_R, 1), lambda i: (i, 0)),
        ],
        out_shape=[
            jax.ShapeDtypeStruct((N, NHID), jnp.float32),
            jax.ShapeDtypeStruct((N, 1), jnp.float32),
        ],
    )(deg, x, w1)


def _t_mid(dinv, agg, b, w, d_out):
    def body(dinv_ref, agg_ref, b_ref, w_ref, hs_ref):
        dinv = dinv_ref[...]
        a = (agg_ref[0] + agg_ref[1]) * dinv + b_ref[...]
        h_in = jnp.maximum(a, 0.0)
        h = jnp.dot(h_in, w_ref[...], preferred_element_type=jnp.float32)
        hs_ref[...] = h * dinv

    return pl.pallas_call(
        body,
        grid=(N // _R,),
        in_specs=[
            pl.BlockSpec((_R, 1), lambda i: (i, 0)),
            pl.BlockSpec((NC, _R, NHID), lambda i: (0, i, 0)),
            pl.BlockSpec((1, NHID), lambda i: (0, 0)),
            pl.BlockSpec((NHID, d_out), lambda i: (0, 0)),
        ],
        out_specs=pl.BlockSpec((_R, d_out), lambda i: (i, 0)),
        out_shape=jax.ShapeDtypeStruct((N, d_out), jnp.float32),
    )(dinv, agg, b, w)


def _t_final(dinv, agg, b):
    def body(dinv_ref, agg_ref, b_ref, out_ref):
        dinv = dinv_ref[...]
        z = (agg_ref[0] + agg_ref[1]) * dinv + b_ref[...]
        col = lax.broadcasted_iota(jnp.int32, z.shape, 1)
        z = jnp.where(col < NCLASS, z, -1e30)
        m = jnp.max(z, axis=1, keepdims=True)
        ez = jnp.where(col < NCLASS, jnp.exp(z - m), 0.0)
        lse = jnp.log(jnp.sum(ez, axis=1, keepdims=True))
        out_ref[...] = (z - m - lse)[:, :NCLASS]

    return pl.pallas_call(
        body,
        grid=(N // _R,),
        in_specs=[
            pl.BlockSpec((_R, 1), lambda i: (i, 0)),
            pl.BlockSpec((NC, _R, DPAD), lambda i: (0, i, 0)),
            pl.BlockSpec((1, DPAD), lambda i: (0, 0)),
        ],
        out_specs=pl.BlockSpec((_R, NCLASS), lambda i: (i, 0)),
        out_shape=jax.ShapeDtypeStruct((N, NCLASS), jnp.float32),
    )(dinv, agg, b)


def kernel(x, adj_t, W1, b1, W2, b2, W3, b3):
    src2d = adj_t[0].reshape(NW, CH_PER_W, ECH)
    dst2d = adj_t[1].reshape(NW, CH_PER_W, ECH)
    zeros1d = jnp.zeros((N_PAD1,), jnp.float32)
    zeros2d = jnp.zeros((N, NHID), jnp.float32)
    w3p = jnp.pad(W3, ((0, 0), (0, DPAD - NCLASS)))
    b3p = jnp.pad(b3, (0, DPAD - NCLASS)).reshape(1, DPAD)

    deg = _deg_kernel(dst2d, zeros1d).reshape(NC, N_PAD1, 1)
    hs1, dinv = _t1(deg, x, W1)
    agg1 = _seg_sum(hs1, src2d, dst2d, zeros2d, NHID)
    hs2 = _t_mid(dinv, agg1, b1.reshape(1, NHID), W2, NHID)
    agg2 = _seg_sum(hs2, src2d, dst2d, zeros2d, NHID)
    hs3 = _t_mid(dinv, agg2, b2.reshape(1, NHID), w3p, DPAD)
    agg3 = _seg_sum(
        hs3, src2d, dst2d, jnp.zeros((N, DPAD), jnp.float32), DPAD,
        tc_tiling=DPAD == NHID,
    )
    return _t_final(dinv, agg3, b3p)


# speedup vs baseline: 1.1059x; 1.0207x over previous
import functools

import jax
import jax.numpy as jnp
from jax import lax
from jax.experimental import pallas as pl
from jax.experimental.pallas import tpu as pltpu
from jax.experimental.pallas import tpu_sc as plsc

N = 10000
E_TOTAL = 320000
NFEAT = 128
NHID = 128
NCLASS = 47
DPAD = 48

NC, NS = 2, 16
NW = NC * NS
ECH = 50
CH_PER_W = E_TOTAL // (ECH * NW)
E_PER_W = ECH * CH_PER_W
NBUF = 5
NGRP = CH_PER_W // NBUF
N_PAD1 = 10240
ROWS1 = N_PAD1 // NS
ROWS_A = 624
ROWS_LAST = N - (NS - 1) * ROWS_A
OFF_LAST = (NS - 1) * ROWS_A


def _sc_mesh():
    return plsc.VectorSubcoreMesh(
        core_axis_name="c", subcore_axis_name="s", num_cores=NC, num_subcores=NS
    )


def _deg_kernel(dst2d, zeros1d):
    @functools.partial(
        pl.kernel,
        out_type=jax.ShapeDtypeStruct((NC, N_PAD1), jnp.float32),
        mesh=_sc_mesh(),
        scratch_types=[
            pltpu.VMEM((CH_PER_W, ECH), jnp.int32),
            pltpu.VMEM((64,), jnp.float32),
            pltpu.VMEM_SHARED((N_PAD1,), jnp.float32),
            [pltpu.SemaphoreType.DMA] * NBUF,
        ],
    )
    def k(dst_hbm, zeros_hbm, out_hbm, dst_v, ones_v, acc_sh, ssem):
        c = lax.axis_index("c")
        s = lax.axis_index("s")
        wid = s * NC + c
        r0 = s * ROWS1
        pltpu.sync_copy(
            zeros_hbm.at[pl.ds(r0, ROWS1)], acc_sh.at[pl.ds(r0, ROWS1)]
        )
        pltpu.sync_copy(dst_hbm.at[wid], dst_v)
        for j in range(4):
            ones_v[pl.ds(j * 16, 16)] = jnp.full((16,), 1.0, jnp.float32)
        plsc.subcore_barrier()

        def body(g, carry):
            descs = []
            for b in range(NBUF):
                i = g * NBUF + b
                descs.append(
                    pltpu.async_copy(
                        ones_v.at[pl.ds(0, ECH)], acc_sh.at[dst_v.at[i]],
                        ssem[b], add=True,
                    )
                )
            for d in descs:
                d.wait()
            return carry

        lax.fori_loop(0, NGRP, body, 0)
        plsc.subcore_barrier()
        pltpu.sync_copy(
            acc_sh.at[pl.ds(r0, ROWS1)],
            out_hbm.at[c].at[pl.ds(r0, ROWS1)],
        )

    return k(dst2d, zeros1d)


def _seg_sum(hs, src3d, dst3d, zeros2d, d, tc_tiling=True):
    @functools.partial(
        pl.kernel,
        out_type=jax.ShapeDtypeStruct((NC, N, d), jnp.float32),
        mesh=_sc_mesh(),
        compiler_params=pltpu.CompilerParams(use_tc_tiling_on_sc=tc_tiling),
        scratch_types=[
            [pltpu.VMEM((ECH,), jnp.int32)] * NBUF,
            [pltpu.VMEM((ECH,), jnp.int32)] * NBUF,
            [pltpu.VMEM((ECH,), jnp.int32)] * NBUF,
            [pltpu.VMEM((ECH, d), jnp.float32)] * NBUF,
            pltpu.VMEM_SHARED((N, d), jnp.float32),
            [pltpu.SemaphoreType.DMA] * NBUF,
            [pltpu.SemaphoreType.DMA] * NBUF,
            [pltpu.SemaphoreType.DMA] * NBUF,
            [pltpu.SemaphoreType.DMA] * NBUF,
        ],
    )
    def k(hs_hbm, src_hbm, dst_hbm, zeros_hbm, out_hbm, src_v, dst_va, dst_vb, rows_v, acc_sh, isem, jsem, gsem, ssem):
        c = lax.axis_index("c")
        s = lax.axis_index("s")
        wid = s * NC + c
        r0 = s * ROWS_A

        def _rowcopy(src_ref, dst_ref):
            @pl.when(s != NS - 1)
            def _():
                pltpu.sync_copy(
                    src_ref.at[pl.ds(r0, ROWS_A)], dst_ref.at[pl.ds(r0, ROWS_A)]
                )

            @pl.when(s == NS - 1)
            def _():
                pltpu.sync_copy(
                    src_ref.at[pl.ds(OFF_LAST, ROWS_LAST)],
                    dst_ref.at[pl.ds(OFF_LAST, ROWS_LAST)],
                )

        @pl.when(c == 0)
        def _():
            _rowcopy(hs_hbm, acc_sh)

        @pl.when(c != 0)
        def _():
            _rowcopy(zeros_hbm, acc_sh)

        plsc.subcore_barrier()

        def _issue_chunk(i, b, dst_slot):
            pltpu.async_copy(src_hbm.at[wid].at[i], src_v[b], isem[b])
            pltpu.async_copy(dst_hbm.at[wid].at[i], dst_slot[b], jsem[b])

        def _drain_scatter(b, dst_slot):
            pltpu.make_async_copy(
                rows_v[b], acc_sh.at[dst_slot[b]], ssem[b]
            ).wait()

        def _group(g, dst_cur, dst_nxt):
            gd = []
            for b in range(NBUF):
                @pl.when(g > 0)
                def _():
                    _drain_scatter(b, dst_nxt)

                pltpu.make_async_copy(
                    src_hbm.at[wid].at[g * NBUF + b], src_v[b], isem[b]
                ).wait()
                gd.append(
                    pltpu.async_copy(hs_hbm.at[src_v[b]], rows_v[b], gsem[b])
                )
            for b in range(NBUF):
                gd[b].wait()
                pltpu.make_async_copy(
                    dst_hbm.at[wid].at[g * NBUF + b], dst_cur[b], jsem[b]
                ).wait()
                pltpu.async_copy(
                    rows_v[b], acc_sh.at[dst_cur[b]], ssem[b], add=True
                )

                @pl.when(g < NGRP - 1)
                def _():
                    _issue_chunk((g + 1) * NBUF + b, b, dst_nxt)

        for b in range(NBUF):
            _issue_chunk(b, b, dst_va)

        def body(g, carry):
            @pl.when(g % 2 == 0)
            def _():
                _group(g, dst_va, dst_vb)

            @pl.when(g % 2 == 1)
            def _():
                _group(g, dst_vb, dst_va)

            return carry

        lax.fori_loop(0, NGRP, body, 0)
        for b in range(NBUF):
            _drain_scatter(b, dst_vb if (NGRP - 1) % 2 == 1 else dst_va)
        plsc.subcore_barrier()
        _rowcopy(acc_sh, out_hbm.at[c])

    return k(hs, src3d, dst3d, zeros2d)


_R = 5000


def _t1(deg, x, w1):
    def body(deg_ref, x_ref, w_ref, hs_ref, dinv_ref):
        d = deg_ref[0] + deg_ref[1] + 1.0
        dinv = lax.rsqrt(d)
        h = jnp.dot(x_ref[...], w_ref[...], preferred_element_type=jnp.float32)
        hs_ref[...] = h * dinv
        dinv_ref[...] = dinv

    return pl.pallas_call(
        body,
        grid=(N // _R,),
        in_specs=[
            pl.BlockSpec((NC, _R, 1), lambda i: (0, i, 0)),
            pl.BlockSpec((_R, NFEAT), lambda i: (i, 0)),
            pl.BlockSpec((NFEAT, NHID), lambda i: (0, 0)),
        ],
        out_specs=[
            pl.BlockSpec((_R, NHID), lambda i: (i, 0)),
            pl.BlockSpec((_R, 1), lambda i: (i, 0)),
        ],
        out_shape=[
            jax.ShapeDtypeStruct((N, NHID), jnp.float32),
            jax.ShapeDtypeStruct((N, 1), jnp.float32),
        ],
    )(deg, x, w1)


def _t_mid(dinv, agg, b, w, d_out):
    def body(dinv_ref, agg_ref, b_ref, w_ref, hs_ref):
        dinv = dinv_ref[...]
        a = (agg_ref[0] + agg_ref[1]) * dinv + b_ref[...]
        h_in = jnp.maximum(a, 0.0)
        h = jnp.dot(h_in, w_ref[...], preferred_element_type=jnp.float32)
        hs_ref[...] = h * dinv

    return pl.pallas_call(
        body,
        grid=(N // _R,),
        in_specs=[
            pl.BlockSpec((_R, 1), lambda i: (i, 0)),
            pl.BlockSpec((NC, _R, NHID), lambda i: (0, i, 0)),
            pl.BlockSpec((1, NHID), lambda i: (0, 0)),
            pl.BlockSpec((NHID, d_out), lambda i: (0, 0)),
        ],
        out_specs=pl.BlockSpec((_R, d_out), lambda i: (i, 0)),
        out_shape=jax.ShapeDtypeStruct((N, d_out), jnp.float32),
    )(dinv, agg, b, w)


def _t_final(dinv, agg, b):
    def body(dinv_ref, agg_ref, b_ref, out_ref):
        dinv = dinv_ref[...]
        z = (agg_ref[0] + agg_ref[1]) * dinv + b_ref[...]
        col = lax.broadcasted_iota(jnp.int32, z.shape, 1)
        z = jnp.where(col < NCLASS, z, -1e30)
        m = jnp.max(z, axis=1, keepdims=True)
        ez = jnp.where(col < NCLASS, jnp.exp(z - m), 0.0)
        lse = jnp.log(jnp.sum(ez, axis=1, keepdims=True))
        out_ref[...] = (z - m - lse)[:, :NCLASS]

    return pl.pallas_call(
        body,
        grid=(N // _R,),
        in_specs=[
            pl.BlockSpec((_R, 1), lambda i: (i, 0)),
            pl.BlockSpec((NC, _R, DPAD), lambda i: (0, i, 0)),
            pl.BlockSpec((1, DPAD), lambda i: (0, 0)),
        ],
        out_specs=pl.BlockSpec((_R, NCLASS), lambda i: (i, 0)),
        out_shape=jax.ShapeDtypeStruct((N, NCLASS), jnp.float32),
    )(dinv, agg, b)


def kernel(x, adj_t, W1, b1, W2, b2, W3, b3):
    src2d = adj_t[0].reshape(NW, CH_PER_W, ECH)
    dst2d = adj_t[1].reshape(NW, CH_PER_W, ECH)
    zeros1d = jnp.zeros((N_PAD1,), jnp.float32)
    zeros2d = jnp.zeros((N, NHID), jnp.float32)
    w3p = jnp.pad(W3, ((0, 0), (0, DPAD - NCLASS)))
    b3p = jnp.pad(b3, (0, DPAD - NCLASS)).reshape(1, DPAD)

    deg = _deg_kernel(dst2d, zeros1d).reshape(NC, N_PAD1, 1)
    hs1, dinv = _t1(deg, x, W1)
    agg1 = _seg_sum(hs1, src2d, dst2d, zeros2d, NHID)
    hs2 = _t_mid(dinv, agg1, b1.reshape(1, NHID), W2, NHID)
    agg2 = _seg_sum(hs2, src2d, dst2d, zeros2d, NHID)
    hs3 = _t_mid(dinv, agg2, b2.reshape(1, NHID), w3p, DPAD)
    agg3 = _seg_sum(
        hs3, src2d, dst2d, jnp.zeros((N, DPAD), jnp.float32), DPAD,
        tc_tiling=DPAD == NHID,
    )
    return _t_final(dinv, agg3, b3p)
